# baseline (speedup 1.0000x reference)
import functools

import jax
import jax.numpy as jnp
from jax import lax
from jax.experimental import pallas as pl
from jax.experimental.pallas import tpu as pltpu
from jax.experimental.pallas import tpu_sc as plsc

_N = 10000
_D = 128
_HD = 64
_NP = 10240
_PAD_ROWS = 64
_NC = 2
_NS = 16
_NW = _NC * _NS
_ROWS = 640
_ROWS_OUT = 400
_TPR = _NP // _NS


def _sc_mesh():
    return plsc.VectorSubcoreMesh(core_axis_name="c", subcore_axis_name="s")


def _deg_body(chunks, dstr_hbm, zeros1_hbm, out_hbm, deg_sp, dstv, ones_v):
    c = lax.axis_index("c")
    s = lax.axis_index("s")
    w = s * _NC + c
    r0 = s * _TPR
    pltpu.sync_copy(zeros1_hbm.at[pl.ds(r0, _TPR)], deg_sp.at[pl.ds(r0, _TPR)])
    for j in range(8):
        ones_v[pl.ds(16 * j, 16)] = jnp.ones((16,), jnp.float32)
    pltpu.sync_copy(dstr_hbm.at[w], dstv)
    plsc.subcore_barrier()

    def body(k, carry):
        pltpu.sync_copy(ones_v, deg_sp.at[dstv.at[k]], add=True)
        return carry

    lax.fori_loop(0, chunks, body, 0)
    plsc.subcore_barrier()
    pltpu.sync_copy(deg_sp.at[pl.ds(r0, _TPR)], out_hbm.at[c].at[pl.ds(r0, _TPR)])


def _make_deg_kernel(chunks):
    return pl.kernel(
        functools.partial(_deg_body, chunks),
        out_type=jax.ShapeDtypeStruct((_NC, _NP), jnp.float32),
        mesh=_sc_mesh(),
        scratch_types=[
            pltpu.VMEM_SHARED((_NP,), jnp.float32),
            pltpu.VMEM((chunks, 128), jnp.int32),
            pltpu.VMEM((128,), jnp.float32),
        ],
    )


def _agg_body(chunks, hp2_hbm, srcr_hbm, dstr_hbm, zeros3_hbm, out_hbm,
              agg_sp, srcv, dstv, b0, b1, b2, b3,
              ga, gb, sa, sb, zsem):
    c = lax.axis_index("c")
    s = lax.axis_index("s")
    r0 = s * _TPR
    zcp = pltpu.async_copy(zeros3_hbm.at[pl.ds(r0, _TPR)],
                           agg_sp.at[pl.ds(r0, _TPR)], zsem)
    pltpu.sync_copy(srcr_hbm.at[s], srcv)
    pltpu.sync_copy(dstr_hbm.at[s], dstv)
    zcp.wait()
    plsc.subcore_barrier()

    quads = chunks // 4

    def gather(k, buf, sem):
        return pltpu.async_copy(hp2_hbm.at[c].at[srcv.at[k]], buf, sem)

    def scatter(k, buf, sem):
        return pltpu.async_copy(buf, agg_sp.at[dstv.at[k]], sem, add=True)

    gather(0, b0, ga)
    gather(1, b1, ga)

    def body(q, carry):
        k = 4 * q
        pltpu.make_async_copy(hp2_hbm.at[c].at[srcv.at[k]], b0, ga).wait()
        pltpu.make_async_copy(hp2_hbm.at[c].at[srcv.at[k + 1]], b1, ga).wait()
        gather(k + 2, b2, gb)
        gather(k + 3, b3, gb)
        scatter(k, b0, sa)
        scatter(k + 1, b1, sa)
        pltpu.make_async_copy(hp2_hbm.at[c].at[srcv.at[k + 2]], b2, gb).wait()
        pltpu.make_async_copy(hp2_hbm.at[c].at[srcv.at[k + 3]], b3, gb).wait()
        pltpu.make_async_copy(b0, agg_sp.at[dstv.at[k]], sa).wait()
        pltpu.make_async_copy(b1, agg_sp.at[dstv.at[k + 1]], sa).wait()

        @pl.when(q + 1 < quads)
        def _():
            gather(k + 4, b0, ga)
            gather(k + 5, b1, ga)

        scatter(k + 2, b2, sb)
        scatter(k + 3, b3, sb)
        pltpu.make_async_copy(b2, agg_sp.at[dstv.at[k + 2]], sb).wait()
        pltpu.make_async_copy(b3, agg_sp.at[dstv.at[k + 3]], sb).wait()
        return carry

    lax.fori_loop(0, quads, body, 0)
    plsc.subcore_barrier()
    pltpu.sync_copy(agg_sp.at[pl.ds(r0, _TPR)],
                    out_hbm.at[c].at[pl.ds(r0, _TPR)])


def _make_agg_kernel(chunks):
    return pl.kernel(
        functools.partial(_agg_body, chunks),
        out_type=jax.ShapeDtypeStruct((_NC, _NP, _HD), jnp.float32),
        mesh=_sc_mesh(),
        compiler_params=pltpu.CompilerParams(use_tc_tiling_on_sc=False),
        scratch_types=[
            pltpu.VMEM_SHARED((_NP, _HD), jnp.float32),
            pltpu.VMEM((chunks, 128), jnp.int32),
            pltpu.VMEM((chunks, 128), jnp.int32),
            pltpu.VMEM((128, _HD), jnp.float32),
            pltpu.VMEM((128, _HD), jnp.float32),
            pltpu.VMEM((128, _HD), jnp.float32),
            pltpu.VMEM((128, _HD), jnp.float32),
            pltpu.SemaphoreType.DMA,
            pltpu.SemaphoreType.DMA,
            pltpu.SemaphoreType.DMA,
            pltpu.SemaphoreType.DMA,
            pltpu.SemaphoreType.DMA,
        ],
    )


def _mm1_body(x_ref, w_ref, d0_ref, d1_ref, hp_ref, dis_ref):
    dis = lax.rsqrt(1.0 + d0_ref[...] + d1_ref[...])
    h = jnp.dot(x_ref[...], w_ref[...], preferred_element_type=jnp.float32)
    hp = h * dis
    hp_ref[0] = hp[:, :_HD]
    hp_ref[1] = hp[:, _HD:]
    dis_ref[...] = dis


def _mm1(xp, w, d0, d1):
    grid = (_NP // _ROWS,)
    return pl.pallas_call(
        _mm1_body,
        grid=grid,
        in_specs=[
            pl.BlockSpec((_ROWS, _D), lambda i: (i, 0)),
            pl.BlockSpec((_D, _D), lambda i: (0, 0)),
            pl.BlockSpec((_ROWS, 1), lambda i: (i, 0)),
            pl.BlockSpec((_ROWS, 1), lambda i: (i, 0)),
        ],
        out_specs=[
            pl.BlockSpec((_NC, _ROWS, _HD), lambda i: (0, i, 0)),
            pl.BlockSpec((_ROWS, 1), lambda i: (i, 0)),
        ],
        out_shape=[
            jax.ShapeDtypeStruct((_NC, _NP, _HD), jnp.float32),
            jax.ShapeDtypeStruct((_NP, 1), jnp.float32),
        ],
    )(xp, w, d0, d1)


def _fused_body(aggp_ref, hp_ref, dis_ref, b_ref, w_ref, o_ref):
    dis = dis_ref[...]
    agg = jnp.concatenate([aggp_ref[0], aggp_ref[1]], axis=-1)
    hp = jnp.concatenate([hp_ref[0], hp_ref[1]], axis=-1)
    xl = jnp.maximum(dis * (agg + hp) + b_ref[...], 0.0)
    h = jnp.dot(xl, w_ref[...], preferred_element_type=jnp.float32)
    hpn = h * dis
    o_ref[0] = hpn[:, :_HD]
    o_ref[1] = hpn[:, _HD:]


def _fused(aggp, hp, dis, b, w):
    grid = (_NP // _ROWS,)
    return pl.pallas_call(
        _fused_body,
        grid=grid,
        in_specs=[
            pl.BlockSpec((_NC, _ROWS, _HD), lambda i: (0, i, 0)),
            pl.BlockSpec((_NC, _ROWS, _HD), lambda i: (0, i, 0)),
            pl.BlockSpec((_ROWS, 1), lambda i: (i, 0)),
            pl.BlockSpec((1, _D), lambda i: (0, 0)),
            pl.BlockSpec((_D, _D), lambda i: (0, 0)),
        ],
        out_specs=pl.BlockSpec((_NC, _ROWS, _HD), lambda i: (0, i, 0)),
        out_shape=jax.ShapeDtypeStruct((_NC, _NP, _HD), jnp.float32),
    )(aggp, hp, dis, b, w)


def _epi_body(aggp_ref, hp_ref, dis_ref, b_ref, o_ref):
    agg = jnp.concatenate([aggp_ref[0], aggp_ref[1]], axis=-1)
    hp = jnp.concatenate([hp_ref[0], hp_ref[1]], axis=-1)
    o_ref[...] = jnp.maximum(
        dis_ref[...] * (agg + hp) + b_ref[...], 0.0)


def _epi(aggp, hp, dis, b):
    grid = (_N // _ROWS_OUT,)
    return pl.pallas_call(
        _epi_body,
        grid=grid,
        in_specs=[
            pl.BlockSpec((_NC, _ROWS_OUT, _HD), lambda i: (0, i, 0)),
            pl.BlockSpec((_NC, _ROWS_OUT, _HD), lambda i: (0, i, 0)),
            pl.BlockSpec((_ROWS_OUT, 1), lambda i: (i, 0)),
            pl.BlockSpec((1, _D), lambda i: (0, 0)),
        ],
        out_specs=pl.BlockSpec((_ROWS_OUT, _D), lambda i: (i, 0)),
        out_shape=jax.ShapeDtypeStruct((_N, _D), jnp.float32),
    )(aggp, hp, dis, b)


def kernel(x, edge_index, W0, b0, W1, b1, W2, b2):
    e = edge_index.shape[1]
    chunk_total = -(-e // (_NW * 128))
    chunk_total = -(-chunk_total // 4) * 4
    ep = chunk_total * _NW * 128
    ch16 = chunk_total * 2

    src = edge_index[0]
    dst = edge_index[1]
    padi = _N + (jnp.arange(ep - e, dtype=jnp.int32) % _PAD_ROWS)
    srcf = jnp.concatenate([src, padi])
    dstf = jnp.concatenate([dst, padi])
    srcp16 = srcf.reshape(_NS, ch16, 128)
    dstp16 = dstf.reshape(_NS, ch16, 128)
    dstp32 = dstf.reshape(_NW, chunk_total, 128)

    zeros1 = jnp.zeros((_NP,), jnp.float32)
    zeros3 = jnp.zeros((_NP, _HD), jnp.float32)
    xp = jnp.pad(x, ((0, _NP - _N), (0, 0)))

    degp = _make_deg_kernel(chunk_total)(dstp32, zeros1)
    d0 = degp[0][:, None]
    d1 = degp[1][:, None]

    agg_k = _make_agg_kernel(ch16)

    hp, dis = _mm1(xp, W0, d0, d1)
    for b, w_next in ((b0, W1), (b1, W2)):
        aggp = agg_k(hp, srcp16, dstp16, zeros3)
        hp = _fused(aggp, hp, dis, b.reshape(1, _D), w_next)
    aggp = agg_k(hp, srcp16, dstp16, zeros3)
    return _epi(aggp, hp, dis, b2.reshape(1, _D))


# speedup vs baseline: 1.2169x; 1.2169x over previous
import functools

import jax
import jax.numpy as jnp
from jax import lax
from jax.experimental import pallas as pl
from jax.experimental.pallas import tpu as pltpu
from jax.experimental.pallas import tpu_sc as plsc

_N = 10000
_D = 128
_NP = 10240
_PAD_ROWS = 64
_NC = 2
_NS = 16
_NW = _NC * _NS
_ROWS = 640
_ROWS_OUT = 400
_TPR = _NP // _NS


def _sc_mesh():
    return plsc.VectorSubcoreMesh(core_axis_name="c", subcore_axis_name="s")


def _deg_body(chunks, dstr_hbm, zeros1_hbm, out_hbm, deg_sp, dstv, ones_v):
    c = lax.axis_index("c")
    s = lax.axis_index("s")
    w = s * _NC + c
    r0 = s * _TPR
    pltpu.sync_copy(zeros1_hbm.at[pl.ds(r0, _TPR)], deg_sp.at[pl.ds(r0, _TPR)])
    for j in range(8):
        ones_v[pl.ds(16 * j, 16)] = jnp.ones((16,), jnp.float32)
    pltpu.sync_copy(dstr_hbm.at[w], dstv)
    plsc.subcore_barrier()

    def body(k, carry):
        pltpu.sync_copy(ones_v, deg_sp.at[dstv.at[k]], add=True)
        return carry

    lax.fori_loop(0, chunks, body, 0)
    plsc.subcore_barrier()
    pltpu.sync_copy(deg_sp.at[pl.ds(r0, _TPR)], out_hbm.at[c].at[pl.ds(r0, _TPR)])


def _make_deg_kernel(chunks):
    return pl.kernel(
        functools.partial(_deg_body, chunks),
        out_type=jax.ShapeDtypeStruct((_NC, _NP), jnp.float32),
        mesh=_sc_mesh(),
        scratch_types=[
            pltpu.VMEM_SHARED((_NP,), jnp.float32),
            pltpu.VMEM((chunks, 128), jnp.int32),
            pltpu.VMEM((128,), jnp.float32),
        ],
    )


_CPR = 40


def _agg_body(chunks, hp_hbm, srcr_hbm, dstr_hbm, zeros2_hbm, out_hbm,
              agg_sp, srcv, dstv, buf0, buf1, gsem0, gsem1, zsem):
    c = lax.axis_index("c")
    s = lax.axis_index("s")
    w = s * _NC + c
    r0 = s * _TPR
    zcp = pltpu.async_copy(zeros2_hbm.at[pl.ds(r0, _TPR)],
                           agg_sp.at[pl.ds(r0, _TPR)], zsem)
    rounds = chunks // _CPR
    half = _CPR // 2

    def round_body(r, carry):
        pltpu.sync_copy(srcr_hbm.at[w].at[pl.ds(r * _CPR, _CPR)], srcv)
        pltpu.sync_copy(dstr_hbm.at[w].at[pl.ds(r * _CPR, _CPR)], dstv)
        pltpu.async_copy(hp_hbm.at[srcv.at[0]], buf0, gsem0)

        @pl.when(r == 0)
        def _():
            pltpu.make_async_copy(zeros2_hbm.at[pl.ds(r0, _TPR)],
                                  agg_sp.at[pl.ds(r0, _TPR)], zsem).wait()
            plsc.subcore_barrier()

        def body(j, carry2):
            k0 = 2 * j
            pltpu.async_copy(hp_hbm.at[srcv.at[k0 + 1]], buf1, gsem1)
            pltpu.make_async_copy(hp_hbm.at[srcv.at[k0]], buf0, gsem0).wait()
            pltpu.sync_copy(buf0, agg_sp.at[dstv.at[k0]], add=True)

            @pl.when(j + 1 < half)
            def _():
                pltpu.async_copy(hp_hbm.at[srcv.at[k0 + 2]], buf0, gsem0)

            pltpu.make_async_copy(hp_hbm.at[srcv.at[k0 + 1]], buf1, gsem1).wait()
            pltpu.sync_copy(buf1, agg_sp.at[dstv.at[k0 + 1]], add=True)
            return carry2

        lax.fori_loop(0, half, body, 0)
        return carry

    lax.fori_loop(0, rounds, round_body, 0)
    plsc.subcore_barrier()
    pltpu.sync_copy(agg_sp.at[pl.ds(r0, _TPR)], out_hbm.at[c].at[pl.ds(r0, _TPR)])


def _make_agg_kernel(chunks):
    return pl.kernel(
        functools.partial(_agg_body, chunks),
        out_type=jax.ShapeDtypeStruct((_NC, _NP, _D), jnp.float32),
        mesh=_sc_mesh(),
        scratch_types=[
            pltpu.VMEM_SHARED((_NP, _D), jnp.float32),
            pltpu.VMEM((_CPR, 128), jnp.int32),
            pltpu.VMEM((_CPR, 128), jnp.int32),
            pltpu.VMEM((128, _D), jnp.float32),
            pltpu.VMEM((128, _D), jnp.float32),
            pltpu.SemaphoreType.DMA,
            pltpu.SemaphoreType.DMA,
            pltpu.SemaphoreType.DMA,
        ],
    )


def _mm_raw_body(x_ref, w_ref, o_ref):
    o_ref[...] = jnp.dot(x_ref[...], w_ref[...],
                         preferred_element_type=jnp.float32)


def _mm_raw(xp, w):
    grid = (_NP // _ROWS,)
    return pl.pallas_call(
        _mm_raw_body,
        grid=grid,
        in_specs=[
            pl.BlockSpec((_ROWS, _D), lambda i: (i, 0)),
            pl.BlockSpec((_D, _D), lambda i: (0, 0)),
        ],
        out_specs=pl.BlockSpec((_ROWS, _D), lambda i: (i, 0)),
        out_shape=jax.ShapeDtypeStruct((_NP, _D), jnp.float32),
    )(xp, w)


def _scale_body(h_ref, d0_ref, d1_ref, hp_ref, dis_ref):
    dis = lax.rsqrt(1.0 + d0_ref[...] + d1_ref[...])
    hp_ref[...] = h_ref[...] * dis
    dis_ref[...] = dis


def _scale(h, d0, d1):
    grid = (_NP // _ROWS,)
    return pl.pallas_call(
        _scale_body,
        grid=grid,
        in_specs=[
            pl.BlockSpec((_ROWS, _D), lambda i: (i, 0)),
            pl.BlockSpec((_ROWS, 1), lambda i: (i, 0)),
            pl.BlockSpec((_ROWS, 1), lambda i: (i, 0)),
        ],
        out_specs=[
            pl.BlockSpec((_ROWS, _D), lambda i: (i, 0)),
            pl.BlockSpec((_ROWS, 1), lambda i: (i, 0)),
        ],
        out_shape=[
            jax.ShapeDtypeStruct((_NP, _D), jnp.float32),
            jax.ShapeDtypeStruct((_NP, 1), jnp.float32),
        ],
    )(h, d0, d1)


def _fused_body(aggp_ref, hp_ref, dis_ref, b_ref, w_ref, o_ref):
    dis = dis_ref[...]
    xl = jnp.maximum(
        dis * (aggp_ref[0] + aggp_ref[1] + hp_ref[...]) + b_ref[...], 0.0)
    h = jnp.dot(xl, w_ref[...], preferred_element_type=jnp.float32)
    o_ref[...] = h * dis


def _fused(aggp, hp, dis, b, w):
    grid = (_NP // _ROWS,)
    return pl.pallas_call(
        _fused_body,
        grid=grid,
        in_specs=[
            pl.BlockSpec((_NC, _ROWS, _D), lambda i: (0, i, 0)),
            pl.BlockSpec((_ROWS, _D), lambda i: (i, 0)),
            pl.BlockSpec((_ROWS, 1), lambda i: (i, 0)),
            pl.BlockSpec((1, _D), lambda i: (0, 0)),
            pl.BlockSpec((_D, _D), lambda i: (0, 0)),
        ],
        out_specs=pl.BlockSpec((_ROWS, _D), lambda i: (i, 0)),
        out_shape=jax.ShapeDtypeStruct((_NP, _D), jnp.float32),
    )(aggp, hp, dis, b, w)


def _epi_body(aggp_ref, hp_ref, dis_ref, b_ref, o_ref):
    o_ref[...] = jnp.maximum(
        dis_ref[...] * (aggp_ref[0] + aggp_ref[1] + hp_ref[...]) + b_ref[...],
        0.0)


def _epi(aggp, hp, dis, b):
    grid = (_N // _ROWS_OUT,)
    return pl.pallas_call(
        _epi_body,
        grid=grid,
        in_specs=[
            pl.BlockSpec((_NC, _ROWS_OUT, _D), lambda i: (0, i, 0)),
            pl.BlockSpec((_ROWS_OUT, _D), lambda i: (i, 0)),
            pl.BlockSpec((_ROWS_OUT, 1), lambda i: (i, 0)),
            pl.BlockSpec((1, _D), lambda i: (0, 0)),
        ],
        out_specs=pl.BlockSpec((_ROWS_OUT, _D), lambda i: (i, 0)),
        out_shape=jax.ShapeDtypeStruct((_N, _D), jnp.float32),
    )(aggp, hp, dis, b)


def kernel(x, edge_index, W0, b0, W1, b1, W2, b2):
    e = edge_index.shape[1]
    chunk_total = -(-e // (_NW * 128))
    chunk_total = -(-chunk_total // _CPR) * _CPR
    ep = chunk_total * _NW * 128

    src = edge_index[0]
    dst = edge_index[1]
    padi = _N + (jnp.arange(ep - e, dtype=jnp.int32) % _PAD_ROWS)
    srcp = jnp.concatenate([src, padi]).reshape(_NW, chunk_total, 128)
    dstp = jnp.concatenate([dst, padi]).reshape(_NW, chunk_total, 128)

    zeros1 = jnp.zeros((_NP,), jnp.float32)
    zeros2 = jnp.zeros((_NP, _D), jnp.float32)
    xp = jnp.pad(x, ((0, _NP - _N), (0, 0)))

    degp = _make_deg_kernel(chunk_total)(dstp, zeros1)
    h1 = _mm_raw(xp, W0)
    d0 = degp[0][:, None]
    d1 = degp[1][:, None]

    agg_k = _make_agg_kernel(chunk_total)

    hp, dis = _scale(h1, d0, d1)
    for b, w_next in ((b0, W1), (b1, W2)):
        aggp = agg_k(hp, srcp, dstp, zeros2)
        hp = _fused(aggp, hp, dis, b.reshape(1, _D), w_next)
    aggp = agg_k(hp, srcp, dstp, zeros2)
    return _epi(aggp, hp, dis, b2.reshape(1, _D))


# speedup vs baseline: 1.2216x; 1.0039x over previous
import functools

import jax
import jax.numpy as jnp
from jax import lax
from jax.experimental import pallas as pl
from jax.experimental.pallas import tpu as pltpu
from jax.experimental.pallas import tpu_sc as plsc

_N = 10000
_D = 128
_NP = 10240
_PAD_ROWS = 64
_NC = 2
_NS = 16
_NW = _NC * _NS
_ROWS = 640
_ROWS_OUT = 400
_TPR = _NP // _NS


def _sc_mesh():
    return plsc.VectorSubcoreMesh(core_axis_name="c", subcore_axis_name="s")


def _deg_body(chunks, dstr_hbm, zeros1_hbm, out_hbm, deg_sp, dstv, ones_v):
    c = lax.axis_index("c")
    s = lax.axis_index("s")
    w = s * _NC + c
    r0 = s * _TPR
    pltpu.sync_copy(zeros1_hbm.at[pl.ds(r0, _TPR)], deg_sp.at[pl.ds(r0, _TPR)])
    for j in range(8):
        ones_v[pl.ds(16 * j, 16)] = jnp.ones((16,), jnp.float32)
    pltpu.sync_copy(dstr_hbm.at[w], dstv)
    plsc.subcore_barrier()

    def body(k, carry):
        pltpu.sync_copy(ones_v, deg_sp.at[dstv.at[k]], add=True)
        return carry

    lax.fori_loop(0, chunks, body, 0)
    plsc.subcore_barrier()
    pltpu.sync_copy(deg_sp.at[pl.ds(r0, _TPR)], out_hbm.at[c].at[pl.ds(r0, _TPR)])


def _make_deg_kernel(chunks):
    return pl.kernel(
        functools.partial(_deg_body, chunks),
        out_type=jax.ShapeDtypeStruct((_NC, _NP), jnp.float32),
        mesh=_sc_mesh(),
        scratch_types=[
            pltpu.VMEM_SHARED((_NP,), jnp.float32),
            pltpu.VMEM((chunks, 128), jnp.int32),
            pltpu.VMEM((128,), jnp.float32),
        ],
    )


_CPR = 40


def _agg_body(chunks, hp_hbm, srcr_hbm, dstr_hbm, zeros2_hbm, out_hbm,
              agg_sp, srcv, dstv, buf0, buf1, gsem0, gsem1, zsem):
    c = lax.axis_index("c")
    s = lax.axis_index("s")
    w = s * _NC + c
    r0 = s * _TPR
    zcp = pltpu.async_copy(zeros2_hbm.at[pl.ds(r0, _TPR)],
                           agg_sp.at[pl.ds(r0, _TPR)], zsem)
    rounds = chunks // _CPR
    half = _CPR // 2

    for r in range(rounds):
        pltpu.sync_copy(srcr_hbm.at[w].at[pl.ds(r * _CPR, _CPR)], srcv)
        pltpu.sync_copy(dstr_hbm.at[w].at[pl.ds(r * _CPR, _CPR)], dstv)
        pltpu.async_copy(hp_hbm.at[srcv.at[0]], buf0, gsem0)

        if r == 0:
            pltpu.make_async_copy(zeros2_hbm.at[pl.ds(r0, _TPR)],
                                  agg_sp.at[pl.ds(r0, _TPR)], zsem).wait()
            plsc.subcore_barrier()

        def body(j, carry2):
            k0 = 2 * j
            pltpu.async_copy(hp_hbm.at[srcv.at[k0 + 1]], buf1, gsem1)
            pltpu.make_async_copy(hp_hbm.at[srcv.at[k0]], buf0, gsem0).wait()
            pltpu.sync_copy(buf0, agg_sp.at[dstv.at[k0]], add=True)

            @pl.when(j + 1 < half)
            def _():
                pltpu.async_copy(hp_hbm.at[srcv.at[k0 + 2]], buf0, gsem0)

            pltpu.make_async_copy(hp_hbm.at[srcv.at[k0 + 1]], buf1, gsem1).wait()
            pltpu.sync_copy(buf1, agg_sp.at[dstv.at[k0 + 1]], add=True)
            return carry2

        lax.fori_loop(0, half, body, 0)
    plsc.subcore_barrier()
    pltpu.sync_copy(agg_sp.at[pl.ds(r0, _TPR)], out_hbm.at[c].at[pl.ds(r0, _TPR)])


def _make_agg_kernel(chunks):
    return pl.kernel(
        functools.partial(_agg_body, chunks),
        out_type=jax.ShapeDtypeStruct((_NC, _NP, _D), jnp.float32),
        mesh=_sc_mesh(),
        scratch_types=[
            pltpu.VMEM_SHARED((_NP, _D), jnp.float32),
            pltpu.VMEM((_CPR, 128), jnp.int32),
            pltpu.VMEM((_CPR, 128), jnp.int32),
            pltpu.VMEM((128, _D), jnp.float32),
            pltpu.VMEM((128, _D), jnp.float32),
            pltpu.SemaphoreType.DMA,
            pltpu.SemaphoreType.DMA,
            pltpu.SemaphoreType.DMA,
        ],
    )


def _mm_raw_body(x_ref, w_ref, o_ref):
    o_ref[...] = jnp.dot(x_ref[...], w_ref[...],
                         preferred_element_type=jnp.float32)


def _mm_raw(xp, w):
    grid = (_NP // _ROWS,)
    return pl.pallas_call(
        _mm_raw_body,
        grid=grid,
        in_specs=[
            pl.BlockSpec((_ROWS, _D), lambda i: (i, 0)),
            pl.BlockSpec((_D, _D), lambda i: (0, 0)),
        ],
        out_specs=pl.BlockSpec((_ROWS, _D), lambda i: (i, 0)),
        out_shape=jax.ShapeDtypeStruct((_NP, _D), jnp.float32),
    )(xp, w)


def _scale_body(h_ref, d0_ref, d1_ref, hp_ref, dis_ref):
    dis = lax.rsqrt(1.0 + d0_ref[...] + d1_ref[...])
    hp_ref[...] = h_ref[...] * dis
    dis_ref[...] = dis


def _scale(h, d0, d1):
    grid = (_NP // _ROWS,)
    return pl.pallas_call(
        _scale_body,
        grid=grid,
        in_specs=[
            pl.BlockSpec((_ROWS, _D), lambda i: (i, 0)),
            pl.BlockSpec((_ROWS, 1), lambda i: (i, 0)),
            pl.BlockSpec((_ROWS, 1), lambda i: (i, 0)),
        ],
        out_specs=[
            pl.BlockSpec((_ROWS, _D), lambda i: (i, 0)),
            pl.BlockSpec((_ROWS, 1), lambda i: (i, 0)),
        ],
        out_shape=[
            jax.ShapeDtypeStruct((_NP, _D), jnp.float32),
            jax.ShapeDtypeStruct((_NP, 1), jnp.float32),
        ],
    )(h, d0, d1)


def _fused_body(aggp_ref, hp_ref, dis_ref, b_ref, w_ref, o_ref):
    dis = dis_ref[...]
    xl = jnp.maximum(
        dis * (aggp_ref[0] + aggp_ref[1] + hp_ref[...]) + b_ref[...], 0.0)
    h = jnp.dot(xl, w_ref[...], preferred_element_type=jnp.float32)
    o_ref[...] = h * dis


def _fused(aggp, hp, dis, b, w):
    grid = (_NP // _ROWS,)
    return pl.pallas_call(
        _fused_body,
        grid=grid,
        in_specs=[
            pl.BlockSpec((_NC, _ROWS, _D), lambda i: (0, i, 0)),
            pl.BlockSpec((_ROWS, _D), lambda i: (i, 0)),
            pl.BlockSpec((_ROWS, 1), lambda i: (i, 0)),
            pl.BlockSpec((1, _D), lambda i: (0, 0)),
            pl.BlockSpec((_D, _D), lambda i: (0, 0)),
        ],
        out_specs=pl.BlockSpec((_ROWS, _D), lambda i: (i, 0)),
        out_shape=jax.ShapeDtypeStruct((_NP, _D), jnp.float32),
    )(aggp, hp, dis, b, w)


def _epi_body(aggp_ref, hp_ref, dis_ref, b_ref, o_ref):
    o_ref[...] = jnp.maximum(
        dis_ref[...] * (aggp_ref[0] + aggp_ref[1] + hp_ref[...]) + b_ref[...],
        0.0)


def _epi(aggp, hp, dis, b):
    grid = (_N // _ROWS_OUT,)
    return pl.pallas_call(
        _epi_body,
        grid=grid,
        in_specs=[
            pl.BlockSpec((_NC, _ROWS_OUT, _D), lambda i: (0, i, 0)),
            pl.BlockSpec((_ROWS_OUT, _D), lambda i: (i, 0)),
            pl.BlockSpec((_ROWS_OUT, 1), lambda i: (i, 0)),
            pl.BlockSpec((1, _D), lambda i: (0, 0)),
        ],
        out_specs=pl.BlockSpec((_ROWS_OUT, _D), lambda i: (i, 0)),
        out_shape=jax.ShapeDtypeStruct((_N, _D), jnp.float32),
    )(aggp, hp, dis, b)


def kernel(x, edge_index, W0, b0, W1, b1, W2, b2):
    e = edge_index.shape[1]
    chunk_total = -(-e // (_NW * 128))
    chunk_total = -(-chunk_total // _CPR) * _CPR
    ep = chunk_total * _NW * 128

    src = edge_index[0]
    dst = edge_index[1]
    padi = _N + (jnp.arange(ep - e, dtype=jnp.int32) % _PAD_ROWS)
    srcp = jnp.concatenate([src, padi]).reshape(_NW, chunk_total, 128)
    dstp = jnp.concatenate([dst, padi]).reshape(_NW, chunk_total, 128)

    zeros1 = jnp.zeros((_NP,), jnp.float32)
    zeros2 = jnp.zeros((_NP, _D), jnp.float32)
    xp = jnp.pad(x, ((0, _NP - _N), (0, 0)))

    degp = _make_deg_kernel(chunk_total)(dstp, zeros1)
    h1 = _mm_raw(xp, W0)
    d0 = degp[0][:, None]
    d1 = degp[1][:, None]

    agg_k = _make_agg_kernel(chunk_total)

    hp, dis = _scale(h1, d0, d1)
    for b, w_next in ((b0, W1), (b1, W2)):
        aggp = agg_k(hp, srcp, dstp, zeros2)
        hp = _fused(aggp, hp, dis, b.reshape(1, _D), w_next)
    aggp = agg_k(hp, srcp, dstp, zeros2)
    return _epi(aggp, hp, dis, b2.reshape(1, _D))
